# Initial kernel scaffold; baseline (speedup 1.0000x reference)
import jax
import jax.numpy as jnp
from jax.experimental import pallas as pl


def kernel(x, edge_index, W0, b0, W1, W2, W3, W4, Wjk, bjk, gamma, beta):
    raise NotImplementedError("write your pallas kernel here")


# speedup vs baseline: 9.7948x; 9.7948x over previous
import functools
import math

import jax
import jax.numpy as jnp
from jax import lax
from jax.experimental import pallas as pl
from jax.experimental.pallas import tpu as pltpu
from jax.experimental.pallas import tpu_sc as plsc

N = 10000
E = 320000
D = 128
ALPHA = 0.1
THETA = 0.5
EPS = 1e-5

NC = 2
NS = 16
NW = NC * NS
EPT = E // NW
K = 80
IT = EPT // K
NPAD = 10240
RPT = NPAD // NS
DG = 16

RB = 400
GRID = N // RB

_mesh = plsc.VectorSubcoreMesh(core_axis_name="c", subcore_axis_name="s")


def _fill_vmem(buf, rows, width, value):
    def _row(i, carry):
        r = buf.at[i]
        for j in range(width // 16):
            r[pl.ds(16 * j, 16)] = jnp.full((16,), value, jnp.float32)
        return carry
    lax.fori_loop(0, rows, _row, 0)


def _zero_acc_slice(buf, acc, base):
    _fill_vmem(buf, K, buf.shape[1], 0.0)
    for t in range(RPT // K):
        pltpu.sync_copy(buf, acc.at[pl.ds(base + t * K, K)])


@functools.partial(
    pl.kernel,
    mesh=_mesh,
    out_type=jax.ShapeDtypeStruct((NC, NPAD, D), jnp.float32),
    scratch_types=[
        pltpu.VMEM((IT, K), jnp.int32),
        pltpu.VMEM((IT, K), jnp.int32),
        pltpu.VMEM((K, D), jnp.float32),
        pltpu.VMEM_SHARED((NPAD, D), jnp.float32),
        pltpu.SemaphoreType.DMA,
    ],
)
def _sc_scatter_rows(x_hbm, src_hbm, dst_hbm, out_hbm, sidx, didx, rows,
                     acc, sem):
    c = lax.axis_index("c")
    s = lax.axis_index("s")
    wid = c * NS + s
    base = s * RPT

    _zero_acc_slice(rows, acc, base)

    pltpu.sync_copy(src_hbm.at[wid], sidx)
    pltpu.sync_copy(dst_hbm.at[wid], didx)

    plsc.subcore_barrier()

    def _step(i, carry):
        pltpu.async_copy(x_hbm.at[sidx.at[i]], rows, sem).wait()
        pltpu.sync_copy(rows, acc.at[didx.at[i]], add=True)
        return carry
    lax.fori_loop(0, IT, _step, 0)

    plsc.subcore_barrier()
    pltpu.sync_copy(acc.at[pl.ds(base, RPT)], out_hbm.at[c, pl.ds(base, RPT)])


@functools.partial(
    pl.kernel,
    mesh=_mesh,
    out_type=jax.ShapeDtypeStruct((NC, NPAD, DG), jnp.float32),
    scratch_types=[
        pltpu.VMEM((IT, K), jnp.int32),
        pltpu.VMEM((K, DG), jnp.float32),
        pltpu.VMEM_SHARED((NPAD, DG), jnp.float32),
    ],
)
def _sc_degree(dst_hbm, out_hbm, didx, ones_v, acc):
    c = lax.axis_index("c")
    s = lax.axis_index("s")
    wid = c * NS + s
    base = s * RPT

    _zero_acc_slice(ones_v, acc, base)
    _fill_vmem(ones_v, K, DG, 1.0)

    pltpu.sync_copy(dst_hbm.at[wid], didx)

    plsc.subcore_barrier()

    def _step(i, carry):
        pltpu.sync_copy(ones_v, acc.at[didx.at[i]], add=True)
        return carry
    lax.fori_loop(0, IT, _step, 0)

    plsc.subcore_barrier()
    pltpu.sync_copy(acc.at[pl.ds(base, RPT)], out_hbm.at[c, pl.ds(base, RPT)])


def _ka_body(x_ref, w_ref, degp_ref, h_ref, hp_ref):
    deg = degp_ref[0, :, 0:1] + degp_ref[1, :, 0:1] + 1.0
    dinv = lax.rsqrt(deg)
    h = jnp.dot(x_ref[...], w_ref[...], preferred_element_type=jnp.float32)
    h_ref[...] = h
    hp_ref[...] = h * dinv


def _kb_body(aggp_ref, hp_ref, degp_ref, b0_ref, x0_ref):
    deg = degp_ref[0, :, 0:1] + degp_ref[1, :, 0:1] + 1.0
    dinv = lax.rsqrt(deg)
    agg = aggp_ref[0] + aggp_ref[1] + hp_ref[...]
    x0_ref[...] = agg * dinv + b0_ref[...]


def _kc_body(beta, aggp_ref, x0_ref, w_ref, wjk_ref, jkin_ref, bias_ref,
             jkout_ref, z_ref, stats_ref):
    agg = aggp_ref[0] + aggp_ref[1]
    h = (1.0 - ALPHA) * agg + ALPHA * x0_ref[...]
    z = (1.0 - beta) * h + beta * jnp.dot(
        h, w_ref[...], preferred_element_type=jnp.float32)
    z_ref[...] = z
    jkout_ref[...] = jkin_ref[...] + bias_ref[...] + jnp.dot(
        z, wjk_ref[...], preferred_element_type=jnp.float32)

    @pl.when(pl.program_id(0) == 0)
    def _():
        stats_ref[...] = jnp.zeros_like(stats_ref)

    su = jnp.sum(z, axis=0, keepdims=True)
    sq = jnp.sum(z * z, axis=0, keepdims=True)
    upd = jnp.concatenate([su, sq, jnp.zeros((6, D), jnp.float32)], axis=0)
    stats_ref[...] = stats_ref[...] + upd


def _kbn_body(z_ref, stats_ref, g_ref, b_ref, out_ref):
    mu = stats_ref[0:1, :] * (1.0 / N)
    ms = stats_ref[1:2, :] * (1.0 / N)
    rstd = lax.rsqrt(ms - mu * mu + EPS)
    zn = (z_ref[...] - mu) * rstd * g_ref[...] + b_ref[...]
    out_ref[...] = jnp.maximum(zn, 0.0)


_row = pl.BlockSpec((RB, D), lambda i: (i, 0))
_full = pl.BlockSpec((D, D), lambda i: (0, 0))
_brow = pl.BlockSpec((1, D), lambda i: (0, 0))
_degs = pl.BlockSpec((2, RB, DG), lambda i: (0, i, 0))
_aggs = pl.BlockSpec((2, RB, D), lambda i: (0, i, 0))
_stat = pl.BlockSpec((8, D), lambda i: (0, 0))
_rowD = jax.ShapeDtypeStruct((N, D), jnp.float32)

_call_a = pl.pallas_call(
    _ka_body, grid=(GRID,),
    in_specs=[_row, _full, _degs],
    out_specs=[_row, _row],
    out_shape=[_rowD, _rowD],
)

_call_b = pl.pallas_call(
    _kb_body, grid=(GRID,),
    in_specs=[_aggs, _row, _degs, _brow],
    out_specs=_row,
    out_shape=_rowD,
)

_call_c = [
    pl.pallas_call(
        functools.partial(_kc_body, float(math.log(THETA / (i + 1) + 1.0))),
        grid=(GRID,),
        in_specs=[_aggs, _row, _full, _full, _row, _brow],
        out_specs=[_row, _row, _stat],
        out_shape=[_rowD, _rowD, jax.ShapeDtypeStruct((8, D), jnp.float32)],
    )
    for i in range(4)
]

_call_bn = pl.pallas_call(
    _kbn_body, grid=(GRID,),
    in_specs=[_row, _stat, _brow, _brow],
    out_specs=_row,
    out_shape=_rowD,
)


def kernel(x, edge_index, W0, b0, W1, W2, W3, W4, Wjk, bjk, gamma, beta):
    src = edge_index[0].reshape(NW, IT, K)
    dst = edge_index[1].reshape(NW, IT, K)

    degp = _sc_degree(dst)
    h, hp = _call_a(x, W0, degp)

    aggp = _sc_scatter_rows(hp, src, dst)
    x0 = _call_b(aggp, hp, degp, b0.reshape(1, D))

    zrow = jnp.zeros((1, D), jnp.float32)
    jk = jnp.zeros((N, D), jnp.float32)
    Ws = [W1, W2, W3, W4]
    z_in = x0
    for i in range(4):
        aggp = _sc_scatter_rows(z_in, src, dst)
        bias = bjk.reshape(1, D) if i == 3 else zrow
        jk, z, stats = _call_c[i](aggp, x0, Ws[i],
                                  Wjk[i * D:(i + 1) * D], jk, bias)
        if i < 3:
            z_in = _call_bn(z, stats, gamma[i].reshape(1, D),
                            beta[i].reshape(1, D))
    return jk


# speedup vs baseline: 14.0445x; 1.4339x over previous
import functools
import math

import jax
import jax.numpy as jnp
from jax import lax
from jax.experimental import pallas as pl
from jax.experimental.pallas import tpu as pltpu
from jax.experimental.pallas import tpu_sc as plsc

N = 10000
E = 320000
D = 128
ALPHA = 0.1
THETA = 0.5
EPS = 1e-5

NC = 2
NS = 16
NW = NC * NS
EPT = E // NW
K2 = 128
EPW = 10240
CH = EPW // K2
PADN = EPW - EPT
NPAD = 10240
RPT = NPAD // NS
DG = 16

RB = 400
GRID = N // RB

_mesh = plsc.VectorSubcoreMesh(core_axis_name="c", subcore_axis_name="s")


def _fill_vmem(buf, rows, width, value):
    def _row(i, carry):
        r = buf.at[i]
        for j in range(width // 16):
            r[pl.ds(16 * j, 16)] = jnp.full((16,), value, jnp.float32)
        return carry
    lax.fori_loop(0, rows, _row, 0)


def _zero_acc_slice(buf, acc, base):
    _fill_vmem(buf, K2, buf.shape[1], 0.0)
    for t in range(RPT // K2):
        pltpu.sync_copy(buf, acc.at[pl.ds(base + t * K2, K2)])


@functools.partial(
    pl.kernel,
    mesh=_mesh,
    out_type=jax.ShapeDtypeStruct((NC, NPAD, D), jnp.float32),
    scratch_types=[
        pltpu.VMEM((CH, K2), jnp.int32),
        pltpu.VMEM((K2,), jnp.int32),
        pltpu.VMEM((K2,), jnp.int32),
        pltpu.VMEM((K2,), jnp.int32),
        pltpu.VMEM((K2,), jnp.int32),
        pltpu.VMEM((K2, D), jnp.float32),
        pltpu.VMEM((K2, D), jnp.float32),
        pltpu.VMEM_SHARED((NPAD, D), jnp.float32),
        pltpu.SemaphoreType.DMA,
        pltpu.SemaphoreType.DMA,
        pltpu.SemaphoreType.DMA,
        pltpu.SemaphoreType.DMA,
        pltpu.SemaphoreType.DMA,
        pltpu.SemaphoreType.DMA,
    ],
)
def _sc_scatter_rows(x_hbm, srcf_hbm, dstp_hbm, out_hbm, didxp, si0, si1,
                     si2, si3, rows0, rows1, acc, sg0, sg1, sa0, sa1, sa2,
                     sa3):
    c = lax.axis_index("c")
    s = lax.axis_index("s")
    wid = c * NS + s
    base = s * RPT
    ebase = wid * EPW

    sidx = [si0, si1, si2, si3]
    sems = [sa0, sa1, sa2, sa3]
    rows = [rows0, rows1]
    gsem = [sg0, sg1]

    def icopy(n, b):
        pltpu.async_copy(srcf_hbm.at[pl.ds(ebase + n * K2, K2)], sidx[b],
                         sems[b])

    def iwait(n, b):
        pltpu.make_async_copy(srcf_hbm.at[pl.ds(ebase + n * K2, K2)],
                              sidx[b], sems[b]).wait()

    def gstart(n, ib, rb):
        pltpu.async_copy(x_hbm.at[sidx[ib]], rows[rb], gsem[rb])

    def gwait(n, ib, rb):
        pltpu.make_async_copy(x_hbm.at[sidx[ib]], rows[rb], gsem[rb]).wait()

    def scat(n, rb):
        pltpu.sync_copy(rows[rb], acc.at[didxp.at[n]], add=True)

    _zero_acc_slice(rows0, acc, base)

    pltpu.sync_copy(dstp_hbm.at[wid], didxp)

    plsc.subcore_barrier()

    icopy(0, 0)
    icopy(1, 1)
    icopy(2, 2)
    icopy(3, 3)
    iwait(0, 0)
    gstart(0, 0, 0)

    def _quad(j, carry):
        q = 4 * j
        gwait(q, 0, 0)
        iwait(q + 1, 1)
        gstart(q + 1, 1, 1)
        scat(q, 0)

        @pl.when(q + 4 < CH)
        def _():
            icopy(q + 4, 0)

        gwait(q + 1, 1, 1)
        iwait(q + 2, 2)
        gstart(q + 2, 2, 0)
        scat(q + 1, 1)

        @pl.when(q + 5 < CH)
        def _():
            icopy(q + 5, 1)

        gwait(q + 2, 2, 0)
        iwait(q + 3, 3)
        gstart(q + 3, 3, 1)
        scat(q + 2, 0)

        @pl.when(q + 6 < CH)
        def _():
            icopy(q + 6, 2)

        gwait(q + 3, 3, 1)

        @pl.when(q + 4 < CH)
        def _():
            iwait(q + 4, 0)
            gstart(q + 4, 0, 0)

        scat(q + 3, 1)

        @pl.when(q + 7 < CH)
        def _():
            icopy(q + 7, 3)

        return carry
    lax.fori_loop(0, CH // 4, _quad, 0)

    plsc.subcore_barrier()
    pltpu.sync_copy(acc.at[pl.ds(base, RPT)], out_hbm.at[c, pl.ds(base, RPT)])


@functools.partial(
    pl.kernel,
    mesh=_mesh,
    out_type=jax.ShapeDtypeStruct((NC, NPAD, DG), jnp.float32),
    scratch_types=[
        pltpu.VMEM((CH, K2), jnp.int32),
        pltpu.VMEM((K2, DG), jnp.float32),
        pltpu.VMEM_SHARED((NPAD, DG), jnp.float32),
    ],
)
def _sc_degree(dstp_hbm, out_hbm, didxp, ones_v, acc):
    c = lax.axis_index("c")
    s = lax.axis_index("s")
    wid = c * NS + s
    base = s * RPT

    _zero_acc_slice(ones_v, acc, base)
    _fill_vmem(ones_v, K2, DG, 1.0)

    pltpu.sync_copy(dstp_hbm.at[wid], didxp)

    plsc.subcore_barrier()

    def _step(i, carry):
        pltpu.sync_copy(ones_v, acc.at[didxp.at[i]], add=True)
        return carry
    lax.fori_loop(0, CH, _step, 0)

    plsc.subcore_barrier()
    pltpu.sync_copy(acc.at[pl.ds(base, RPT)], out_hbm.at[c, pl.ds(base, RPT)])


def _ka_body(x_ref, w_ref, degp_ref, h_ref, hp_ref):
    deg = degp_ref[0, :, 0:1] + degp_ref[1, :, 0:1] + 1.0
    dinv = lax.rsqrt(deg)
    h = jnp.dot(x_ref[...], w_ref[...], preferred_element_type=jnp.float32)
    h_ref[...] = h
    hp_ref[...] = h * dinv


def _kb_body(aggp_ref, hp_ref, degp_ref, b0_ref, x0_ref):
    deg = degp_ref[0, :, 0:1] + degp_ref[1, :, 0:1] + 1.0
    dinv = lax.rsqrt(deg)
    agg = aggp_ref[0] + aggp_ref[1] + hp_ref[...]
    x0_ref[...] = agg * dinv + b0_ref[...]


def _kc_body(beta, aggp_ref, x0_ref, w_ref, wjk_ref, jkin_ref, bias_ref,
             jkout_ref, z_ref, stats_ref):
    agg = aggp_ref[0] + aggp_ref[1]
    h = (1.0 - ALPHA) * agg + ALPHA * x0_ref[...]
    z = (1.0 - beta) * h + beta * jnp.dot(
        h, w_ref[...], preferred_element_type=jnp.float32)
    z_ref[...] = z
    jkout_ref[...] = jkin_ref[...] + bias_ref[...] + jnp.dot(
        z, wjk_ref[...], preferred_element_type=jnp.float32)

    @pl.when(pl.program_id(0) == 0)
    def _():
        stats_ref[...] = jnp.zeros_like(stats_ref)

    su = jnp.sum(z, axis=0, keepdims=True)
    sq = jnp.sum(z * z, axis=0, keepdims=True)
    upd = jnp.concatenate([su, sq, jnp.zeros((6, D), jnp.float32)], axis=0)
    stats_ref[...] = stats_ref[...] + upd


def _kbn_body(z_ref, stats_ref, g_ref, b_ref, out_ref):
    mu = stats_ref[0:1, :] * (1.0 / N)
    ms = stats_ref[1:2, :] * (1.0 / N)
    rstd = lax.rsqrt(ms - mu * mu + EPS)
    zn = (z_ref[...] - mu) * rstd * g_ref[...] + b_ref[...]
    out_ref[...] = jnp.maximum(zn, 0.0)


_row = pl.BlockSpec((RB, D), lambda i: (i, 0))
_full = pl.BlockSpec((D, D), lambda i: (0, 0))
_brow = pl.BlockSpec((1, D), lambda i: (0, 0))
_degs = pl.BlockSpec((2, RB, DG), lambda i: (0, i, 0))
_aggs = pl.BlockSpec((2, RB, D), lambda i: (0, i, 0))
_stat = pl.BlockSpec((8, D), lambda i: (0, 0))
_rowD = jax.ShapeDtypeStruct((N, D), jnp.float32)

_call_a = pl.pallas_call(
    _ka_body, grid=(GRID,),
    in_specs=[_row, _full, _degs],
    out_specs=[_row, _row],
    out_shape=[_rowD, _rowD],
)

_call_b = pl.pallas_call(
    _kb_body, grid=(GRID,),
    in_specs=[_aggs, _row, _degs, _brow],
    out_specs=_row,
    out_shape=_rowD,
)

_call_c = [
    pl.pallas_call(
        functools.partial(_kc_body, float(math.log(THETA / (i + 1) + 1.0))),
        grid=(GRID,),
        in_specs=[_aggs, _row, _full, _full, _row, _brow],
        out_specs=[_row, _row, _stat],
        out_shape=[_rowD, _rowD, jax.ShapeDtypeStruct((8, D), jnp.float32)],
    )
    for i in range(4)
]

_call_bn = pl.pallas_call(
    _kbn_body, grid=(GRID,),
    in_specs=[_row, _stat, _brow, _brow],
    out_specs=_row,
    out_shape=_rowD,
)


def kernel(x, edge_index, W0, b0, W1, W2, W3, W4, Wjk, bjk, gamma, beta):
    pada = jnp.arange(PADN, dtype=jnp.int32)
    s2 = edge_index[0].reshape(NW, EPT)
    d2 = edge_index[1].reshape(NW, EPT)
    sp = jnp.concatenate(
        [s2, jnp.broadcast_to(pada[None, :], (NW, PADN))], axis=1)
    dp = jnp.concatenate(
        [d2, jnp.broadcast_to((N + pada)[None, :], (NW, PADN))], axis=1)
    src = sp.reshape(NW * EPW)
    dst = dp.reshape(NW, CH, K2)

    degp = _sc_degree(dst)
    h, hp = _call_a(x, W0, degp)

    aggp = _sc_scatter_rows(hp, src, dst)
    x0 = _call_b(aggp, hp, degp, b0.reshape(1, D))

    zrow = jnp.zeros((1, D), jnp.float32)
    jk = jnp.zeros((N, D), jnp.float32)
    Ws = [W1, W2, W3, W4]
    z_in = x0
    for i in range(4):
        aggp = _sc_scatter_rows(z_in, src, dst)
        bias = bjk.reshape(1, D) if i == 3 else zrow
        jk, z, stats = _call_c[i](aggp, x0, Ws[i],
                                  Wjk[i * D:(i + 1) * D], jk, bias)
        if i < 3:
            z_in = _call_bn(z, stats, gamma[i].reshape(1, D),
                            beta[i].reshape(1, D))
    return jk
